# baseline (speedup 1.0000x reference)
import functools

import jax
import jax.numpy as jnp
from jax import lax
from jax.experimental import pallas as pl
from jax.experimental.pallas import tpu as pltpu
from jax.experimental.pallas import tpu_sc as plsc

N = 10000
E = 160000
D = 256
EMB = 256
OUT = 128

BN = 2000

HWA = 144
CH = 64
NCH = E // CH
CPT = 80
NCHP = 32 * CPT
NSUB = 16
ZR = 624


def _enc_body(x_ref, w1_ref, b1_ref, a2_ref, wg_ref, wa1_ref,
              hw_ref, s2_ref, tacc_ref):
    x = x_ref[...]
    h = jnp.maximum(
        jnp.dot(x, w1_ref[...], preferred_element_type=jnp.float32)
        + b1_ref[...],
        0.0,
    )
    hw = jnp.dot(h, wg_ref[...], preferred_element_type=jnp.float32)
    pad_cols = lax.broadcasted_iota(jnp.int32, (BN, HWA - OUT), 1)
    pad = jnp.where(pad_cols == 0, 1.0, 0.0).astype(jnp.float32)
    hw_ref[...] = jnp.concatenate([hw, pad], axis=1)
    s2_ref[...] = jnp.dot(hw, a2_ref[...], preferred_element_type=jnp.float32)

    @pl.when(pl.program_id(0) == 0)
    def _():
        tacc_ref[...] = jnp.zeros_like(tacc_ref)

    tacc_ref[...] += lax.dot_general(
        x, wa1_ref[...], (((0,), (0,)), ((), ())),
        preferred_element_type=jnp.float32,
    )


def _encoder(x, W1, b1, Wg, a2, Wa1):
    grid = (N // BN,)
    return pl.pallas_call(
        _enc_body,
        grid=grid,
        in_specs=[
            pl.BlockSpec((BN, D), lambda i: (i, 0)),
            pl.BlockSpec((D, EMB), lambda i: (0, 0)),
            pl.BlockSpec((1, EMB), lambda i: (0, 0)),
            pl.BlockSpec((OUT, 2), lambda i: (0, 0)),
            pl.BlockSpec((EMB, OUT), lambda i: (0, 0)),
            pl.BlockSpec((BN, EMB), lambda i: (i, 0)),
        ],
        out_specs=[
            pl.BlockSpec((BN, HWA), lambda i: (i, 0)),
            pl.BlockSpec((BN, 2), lambda i: (i, 0)),
            pl.BlockSpec((D, EMB), lambda i: (0, 0)),
        ],
        out_shape=[
            jax.ShapeDtypeStruct((N, HWA), jnp.float32),
            jax.ShapeDtypeStruct((N, 2), jnp.float32),
            jax.ShapeDtypeStruct((D, EMB), jnp.float32),
        ],
    )(x, W1, b1.reshape(1, EMB), a2, Wg, Wa1)


def _sc_edge(hw_aug, s_src, s_dst, src2d, dst2d):
    mesh = plsc.VectorSubcoreMesh(core_axis_name="c", subcore_axis_name="s")

    @functools.partial(
        pl.kernel,
        mesh=mesh,
        out_type=jax.ShapeDtypeStruct((2, N, HWA), jnp.float32),
        scratch_types=[
            pltpu.VMEM((CPT, CH), jnp.int32),
            pltpu.VMEM((CPT, CH), jnp.int32),
            pltpu.VMEM((3, CH), jnp.float32),
            pltpu.VMEM((3, CH), jnp.float32),
            pltpu.VMEM((3, CH + 16), jnp.float32),
            pltpu.VMEM((3, CH, HWA), jnp.float32),
            pltpu.VMEM_SHARED((N, HWA), jnp.float32),
            pltpu.SemaphoreType.DMA,
            pltpu.SemaphoreType.DMA,
            pltpu.SemaphoreType.DMA,
            pltpu.SemaphoreType.DMA,
            pltpu.SemaphoreType.DMA,
            pltpu.SemaphoreType.DMA,
            pltpu.SemaphoreType.DMA,
            pltpu.SemaphoreType.DMA,
            pltpu.SemaphoreType.DMA,
        ],
        compiler_params=pltpu.CompilerParams(use_tc_tiling_on_sc=False),
    )
    def k(hw_hbm, ssrc_hbm, sdst_hbm, src_hbm, dst_hbm, out_hbm,
          src_idx, dst_idx, ssb, sdb, exb, rows, acc_sh,
          sem_r0, sem_r1, sem_r2, sem_s0, sem_s1, sem_s2,
          sem_w0, sem_w1, sem_w2):
        c = lax.axis_index("c")
        s = lax.axis_index("s")
        w = c * NSUB + s

        def zrow(r, _):
            for kk in range(HWA // 16):
                rows[0, r, pl.ds(kk * 16, 16)] = jnp.zeros((16,), jnp.float32)
            return 0
        lax.fori_loop(0, CH, zrow, 0)

        def zcp(i, _):
            pltpu.sync_copy(rows.at[0].at[pl.ds(0, 48)],
                            acc_sh.at[pl.ds(s * ZR + i * 48, 48)])
            return 0
        lax.fori_loop(0, ZR // 48, zcp, 0)

        @pl.when(s == 0)
        def _():
            pltpu.sync_copy(rows.at[0].at[pl.ds(0, 16)],
                            acc_sh.at[pl.ds(N - 16, 16)])

        pltpu.sync_copy(src_hbm.at[pl.ds(w * CPT, CPT)], src_idx)
        pltpu.sync_copy(dst_hbm.at[pl.ds(w * CPT, CPT)], dst_idx)
        n_real = jnp.clip(NCH - w * CPT, 0, CPT)

        sem_r = [sem_r0, sem_r1, sem_r2]
        sem_s = [sem_s0, sem_s1, sem_s2]
        sem_w = [sem_w0, sem_w1, sem_w2]

        def issue(j, p):
            pltpu.async_copy(ssrc_hbm.at[src_idx.at[j]], ssb.at[p], sem_s[p])
            pltpu.async_copy(sdst_hbm.at[dst_idx.at[j]], sdb.at[p], sem_s[p])
            pltpu.async_copy(hw_hbm.at[src_idx.at[j]], rows.at[p], sem_r[p])

        def wait(j, p):
            pltpu.make_async_copy(ssrc_hbm.at[src_idx.at[j]], ssb.at[p],
                                  sem_s[p]).wait()
            pltpu.make_async_copy(sdst_hbm.at[dst_idx.at[j]], sdb.at[p],
                                  sem_s[p]).wait()
            pltpu.make_async_copy(hw_hbm.at[src_idx.at[j]], rows.at[p],
                                  sem_r[p]).wait()

        plsc.subcore_barrier()

        @pl.when(n_real > 0)
        def _():
            issue(0, 0)

        def process(j, p):
            @pl.when(j >= 2)
            def _():
                pltpu.make_async_copy(
                    rows.at[(p + 1) % 3], acc_sh.at[dst_idx.at[j - 2]],
                    sem_w[(p + 1) % 3]).wait()

            @pl.when(j + 1 < n_real)
            def _():
                issue(j + 1, (p + 1) % 3)

            wait(j, p)

            for g in range(CH // 16):
                sl = pl.ds(g * 16, 16)
                logit = ssb[p, sl] + sdb[p, sl]
                logit = jnp.where(logit >= 0, logit, 0.2 * logit)
                exb[p, sl] = jnp.exp(logit)

            @plsc.parallel_loop(0, CH, 1, unroll=8)
            def _(r):
                e = exb[p, pl.ds(r, 16)][0]
                for kk in range(HWA // 16):
                    sl = pl.ds(kk * 16, 16)
                    rows[p, r, sl] = rows[p, r, sl] * e

            @pl.when(j + 2 < n_real)
            def _():
                pltpu.async_copy(rows.at[p], acc_sh.at[dst_idx.at[j]],
                                 sem_w[p], add=True)

            @pl.when(j + 2 >= n_real)
            def _():
                pltpu.sync_copy(rows.at[p], acc_sh.at[dst_idx.at[j]], add=True)

        def chunk3(j3, _):
            for phase in range(3):
                j = j3 * 3 + phase

                @pl.when(j < n_real)
                def _():
                    process(j, phase)
            return 0
        lax.fori_loop(0, (n_real + 2) // 3, chunk3, 0)

        plsc.subcore_barrier()

        def wout(i, _):
            sl = pl.ds(s * ZR + i * 48, 48)
            pltpu.sync_copy(acc_sh.at[sl], out_hbm.at[c].at[sl])
            return 0
        lax.fori_loop(0, ZR // 48, wout, 0)

        @pl.when(s == 0)
        def _():
            sl = pl.ds(N - 16, 16)
            pltpu.sync_copy(acc_sh.at[sl], out_hbm.at[c].at[sl])

    return k(hw_aug, s_src, s_dst, src2d, dst2d)


def _finx_body(acc_ref, bg_ref, tacc_ref, ba1_ref, wa2_ref, ba2_ref,
               emb_ref, xhat_ref):
    a = acc_ref[0] + acc_ref[1]
    num = a[:, :OUT]
    den = a[:, OUT:OUT + 1]
    embed = num / (den + 1e-16) + bg_ref[...]
    emb_ref[...] = embed
    t1 = jnp.maximum(tacc_ref[...] + ba1_ref[...], 0.0)
    t2 = jnp.dot(t1, wa2_ref[...], preferred_element_type=jnp.float32)
    t2 = t2 + ba2_ref[...]
    xhat_ref[...] = lax.dot_general(
        embed, t2, (((1,), (1,)), ((), ())),
        preferred_element_type=jnp.float32,
    )


def _finalize_xhat(acc2, bg, tacc, ba1, Wa2, ba2):
    grid = (N // BN,)
    return pl.pallas_call(
        _finx_body,
        grid=grid,
        in_specs=[
            pl.BlockSpec((2, BN, HWA), lambda i: (0, i, 0)),
            pl.BlockSpec((1, OUT), lambda i: (0, 0)),
            pl.BlockSpec((D, EMB), lambda i: (0, 0)),
            pl.BlockSpec((1, EMB), lambda i: (0, 0)),
            pl.BlockSpec((EMB, OUT), lambda i: (0, 0)),
            pl.BlockSpec((1, OUT), lambda i: (0, 0)),
        ],
        out_specs=[
            pl.BlockSpec((BN, OUT), lambda i: (i, 0)),
            pl.BlockSpec((BN, D), lambda i: (i, 0)),
        ],
        out_shape=[
            jax.ShapeDtypeStruct((N, OUT), jnp.float32),
            jax.ShapeDtypeStruct((N, D), jnp.float32),
        ],
    )(acc2, bg.reshape(1, OUT), tacc, ba1.reshape(1, EMB), Wa2,
      ba2.reshape(1, OUT))


def _ahat_body(ei_ref, ej_ref, out_ref):
    acc = lax.dot_general(
        ei_ref[...], ej_ref[...], (((1,), (1,)), ((), ())),
        preferred_element_type=jnp.float32,
    )
    out_ref[...] = jax.nn.sigmoid(acc)


def _ahat(embed):
    bi = 2048
    nb = (N + bi - 1) // bi
    grid = (nb, nb)
    return pl.pallas_call(
        _ahat_body,
        grid=grid,
        in_specs=[
            pl.BlockSpec((bi, OUT), lambda i, j: (i, 0)),
            pl.BlockSpec((bi, OUT), lambda i, j: (j, 0)),
        ],
        out_specs=pl.BlockSpec((bi, bi), lambda i, j: (i, j)),
        out_shape=jax.ShapeDtypeStruct((N, N), jnp.float32),
    )(embed, embed)


def _tacc_body(x_ref, wa1_ref, acc_ref):
    @pl.when(pl.program_id(0) == 0)
    def _():
        acc_ref[...] = jnp.zeros_like(acc_ref)

    acc_ref[...] += lax.dot_general(
        x_ref[...], wa1_ref[...], (((0,), (0,)), ((), ())),
        preferred_element_type=jnp.float32,
    )


def _tacc(x, Wa1):
    grid = (N // BN,)
    return pl.pallas_call(
        _tacc_body,
        grid=grid,
        in_specs=[
            pl.BlockSpec((BN, D), lambda i: (i, 0)),
            pl.BlockSpec((BN, EMB), lambda i: (i, 0)),
        ],
        out_specs=pl.BlockSpec((D, EMB), lambda i: (0, 0)),
        out_shape=jax.ShapeDtypeStruct((D, EMB), jnp.float32),
    )(x, Wa1)


def kernel(x, edge_index, W1, b1, Wg, a_src, a_dst, bg, Wa1, ba1, Wa2, ba2):
    a2 = jnp.stack([a_src, a_dst], axis=1)
    hw_aug, s2, tacc = _encoder(x, W1, b1, Wg, a2, Wa1)
    pad = NCHP * CH - E
    src2d = jnp.pad(edge_index[0], (0, pad)).reshape(NCHP, CH)
    dst2d = jnp.pad(edge_index[1], (0, pad)).reshape(NCHP, CH)
    acc2 = _sc_edge(hw_aug, s2[:, 0], s2[:, 1], src2d, dst2d)
    embed, X_hat = _finalize_xhat(acc2, bg, tacc, ba1, Wa2, ba2)
    A_hat = _ahat(embed)
    return (A_hat, X_hat)


# speedup vs baseline: 1.0102x; 1.0102x over previous
import functools

import jax
import jax.numpy as jnp
from jax import lax
from jax.experimental import pallas as pl
from jax.experimental.pallas import tpu as pltpu
from jax.experimental.pallas import tpu_sc as plsc

N = 10000
E = 160000
D = 256
EMB = 256
OUT = 128

BN = 2000

HWA = 144
CH = 64
NCH = E // CH
CPT = 80
NCHP = 32 * CPT
NSUB = 16
ZR = 624


def _enc_body(x_ref, w1_ref, b1_ref, a2_ref, wg_ref, hw_ref, s2_ref):
    x = x_ref[...]
    h = jnp.maximum(
        jnp.dot(x, w1_ref[...], preferred_element_type=jnp.float32)
        + b1_ref[...],
        0.0,
    )
    hw = jnp.dot(h, wg_ref[...], preferred_element_type=jnp.float32)
    pad_cols = lax.broadcasted_iota(jnp.int32, (BN, HWA - OUT), 1)
    pad = jnp.where(pad_cols == 0, 1.0, 0.0).astype(jnp.float32)
    hw_ref[...] = jnp.concatenate([hw, pad], axis=1)
    s2_ref[...] = jnp.dot(hw, a2_ref[...], preferred_element_type=jnp.float32)


def _encoder(x, W1, b1, Wg, a2):
    grid = (N // BN,)
    return pl.pallas_call(
        _enc_body,
        grid=grid,
        in_specs=[
            pl.BlockSpec((BN, D), lambda i: (i, 0)),
            pl.BlockSpec((D, EMB), lambda i: (0, 0)),
            pl.BlockSpec((1, EMB), lambda i: (0, 0)),
            pl.BlockSpec((OUT, 2), lambda i: (0, 0)),
            pl.BlockSpec((EMB, OUT), lambda i: (0, 0)),
        ],
        out_specs=[
            pl.BlockSpec((BN, HWA), lambda i: (i, 0)),
            pl.BlockSpec((BN, 2), lambda i: (i, 0)),
        ],
        out_shape=[
            jax.ShapeDtypeStruct((N, HWA), jnp.float32),
            jax.ShapeDtypeStruct((N, 2), jnp.float32),
        ],
    )(x, W1, b1.reshape(1, EMB), a2, Wg)


def _sc_edge(hw_aug, s_src, s_dst, src2d, dst2d):
    mesh = plsc.VectorSubcoreMesh(core_axis_name="c", subcore_axis_name="s")

    @functools.partial(
        pl.kernel,
        mesh=mesh,
        out_type=jax.ShapeDtypeStruct((2, N, HWA), jnp.float32),
        scratch_types=[
            pltpu.VMEM((CPT, CH), jnp.int32),
            pltpu.VMEM((CPT, CH), jnp.int32),
            pltpu.VMEM((3, CH), jnp.float32),
            pltpu.VMEM((3, CH), jnp.float32),
            pltpu.VMEM((3, CH + 16), jnp.float32),
            pltpu.VMEM((3, CH, HWA), jnp.float32),
            pltpu.VMEM_SHARED((N, HWA), jnp.float32),
            pltpu.SemaphoreType.DMA,
            pltpu.SemaphoreType.DMA,
            pltpu.SemaphoreType.DMA,
            pltpu.SemaphoreType.DMA,
            pltpu.SemaphoreType.DMA,
            pltpu.SemaphoreType.DMA,
            pltpu.SemaphoreType.DMA,
            pltpu.SemaphoreType.DMA,
            pltpu.SemaphoreType.DMA,
        ],
        compiler_params=pltpu.CompilerParams(use_tc_tiling_on_sc=False),
    )
    def k(hw_hbm, ssrc_hbm, sdst_hbm, src_hbm, dst_hbm, out_hbm,
          src_idx, dst_idx, ssb, sdb, exb, rows, acc_sh,
          sem_r0, sem_r1, sem_r2, sem_s0, sem_s1, sem_s2,
          sem_w0, sem_w1, sem_w2):
        c = lax.axis_index("c")
        s = lax.axis_index("s")
        w = c * NSUB + s

        def zrow(r, _):
            for kk in range(HWA // 16):
                rows[0, r, pl.ds(kk * 16, 16)] = jnp.zeros((16,), jnp.float32)
            return 0
        lax.fori_loop(0, CH, zrow, 0)

        def zcp(i, _):
            pltpu.sync_copy(rows.at[0].at[pl.ds(0, 48)],
                            acc_sh.at[pl.ds(s * ZR + i * 48, 48)])
            return 0
        lax.fori_loop(0, ZR // 48, zcp, 0)

        @pl.when(s == 0)
        def _():
            pltpu.sync_copy(rows.at[0].at[pl.ds(0, 16)],
                            acc_sh.at[pl.ds(N - 16, 16)])

        pltpu.sync_copy(src_hbm.at[pl.ds(w * CPT, CPT)], src_idx)
        pltpu.sync_copy(dst_hbm.at[pl.ds(w * CPT, CPT)], dst_idx)
        n_real = jnp.clip(NCH - w * CPT, 0, CPT)

        sem_r = [sem_r0, sem_r1, sem_r2]
        sem_s = [sem_s0, sem_s1, sem_s2]
        sem_w = [sem_w0, sem_w1, sem_w2]

        def issue(j, p):
            pltpu.async_copy(ssrc_hbm.at[src_idx.at[j]], ssb.at[p], sem_s[p])
            pltpu.async_copy(sdst_hbm.at[dst_idx.at[j]], sdb.at[p], sem_s[p])
            pltpu.async_copy(hw_hbm.at[src_idx.at[j]], rows.at[p], sem_r[p])

        def wait(j, p):
            pltpu.make_async_copy(ssrc_hbm.at[src_idx.at[j]], ssb.at[p],
                                  sem_s[p]).wait()
            pltpu.make_async_copy(sdst_hbm.at[dst_idx.at[j]], sdb.at[p],
                                  sem_s[p]).wait()
            pltpu.make_async_copy(hw_hbm.at[src_idx.at[j]], rows.at[p],
                                  sem_r[p]).wait()

        plsc.subcore_barrier()

        @pl.when(n_real > 0)
        def _():
            issue(0, 0)

        def process(j, p):
            @pl.when(j >= 2)
            def _():
                pltpu.make_async_copy(
                    rows.at[(p + 1) % 3], acc_sh.at[dst_idx.at[j - 2]],
                    sem_w[(p + 1) % 3]).wait()

            @pl.when(j + 1 < n_real)
            def _():
                issue(j + 1, (p + 1) % 3)

            wait(j, p)

            for g in range(CH // 16):
                sl = pl.ds(g * 16, 16)
                logit = ssb[p, sl] + sdb[p, sl]
                logit = jnp.where(logit >= 0, logit, 0.2 * logit)
                exb[p, sl] = jnp.exp(logit)

            @plsc.parallel_loop(0, CH, 1, unroll=8)
            def _(r):
                e = exb[p, pl.ds(r, 16)][0]
                for kk in range(HWA // 16):
                    sl = pl.ds(kk * 16, 16)
                    rows[p, r, sl] = rows[p, r, sl] * e

            @pl.when(j + 2 < n_real)
            def _():
                pltpu.async_copy(rows.at[p], acc_sh.at[dst_idx.at[j]],
                                 sem_w[p], add=True)

            @pl.when(j + 2 >= n_real)
            def _():
                pltpu.sync_copy(rows.at[p], acc_sh.at[dst_idx.at[j]], add=True)

        def chunk3(j3, _):
            for phase in range(3):
                j = j3 * 3 + phase

                @pl.when(j < n_real)
                def _():
                    process(j, phase)
            return 0
        lax.fori_loop(0, (n_real + 2) // 3, chunk3, 0)

        plsc.subcore_barrier()

        def wout(i, _):
            sl = pl.ds(s * ZR + i * 48, 48)
            pltpu.sync_copy(acc_sh.at[sl], out_hbm.at[c].at[sl])
            return 0
        lax.fori_loop(0, ZR // 48, wout, 0)

        @pl.when(s == 0)
        def _():
            sl = pl.ds(N - 16, 16)
            pltpu.sync_copy(acc_sh.at[sl], out_hbm.at[c].at[sl])

    return k(hw_aug, s_src, s_dst, src2d, dst2d)


def _finx_body(acc_ref, bg_ref, tacc_ref, ba1_ref, wa2_ref, ba2_ref,
               emb_ref, xhat_ref):
    a = acc_ref[0] + acc_ref[1]
    num = a[:, :OUT]
    den = a[:, OUT:OUT + 1]
    embed = num / (den + 1e-16) + bg_ref[...]
    emb_ref[...] = embed
    t1 = jnp.maximum(tacc_ref[...] + ba1_ref[...], 0.0)
    t2 = jnp.dot(t1, wa2_ref[...], preferred_element_type=jnp.float32)
    t2 = t2 + ba2_ref[...]
    xhat_ref[...] = lax.dot_general(
        embed, t2, (((1,), (1,)), ((), ())),
        preferred_element_type=jnp.float32,
    )


def _finalize_xhat(acc2, bg, tacc, ba1, Wa2, ba2):
    grid = (N // BN,)
    return pl.pallas_call(
        _finx_body,
        grid=grid,
        in_specs=[
            pl.BlockSpec((2, BN, HWA), lambda i: (0, i, 0)),
            pl.BlockSpec((1, OUT), lambda i: (0, 0)),
            pl.BlockSpec((D, EMB), lambda i: (0, 0)),
            pl.BlockSpec((1, EMB), lambda i: (0, 0)),
            pl.BlockSpec((EMB, OUT), lambda i: (0, 0)),
            pl.BlockSpec((1, OUT), lambda i: (0, 0)),
        ],
        out_specs=[
            pl.BlockSpec((BN, OUT), lambda i: (i, 0)),
            pl.BlockSpec((BN, D), lambda i: (i, 0)),
        ],
        out_shape=[
            jax.ShapeDtypeStruct((N, OUT), jnp.float32),
            jax.ShapeDtypeStruct((N, D), jnp.float32),
        ],
    )(acc2, bg.reshape(1, OUT), tacc, ba1.reshape(1, EMB), Wa2,
      ba2.reshape(1, OUT))


def _ahat_body(ei_ref, ej_ref, out_ref):
    acc = lax.dot_general(
        ei_ref[...], ej_ref[...], (((1,), (1,)), ((), ())),
        preferred_element_type=jnp.float32,
    )
    out_ref[...] = jax.nn.sigmoid(acc)


def _ahat(embed):
    bi = 2048
    nb = (N + bi - 1) // bi
    grid = (nb, nb)
    return pl.pallas_call(
        _ahat_body,
        grid=grid,
        in_specs=[
            pl.BlockSpec((bi, OUT), lambda i, j: (i, 0)),
            pl.BlockSpec((bi, OUT), lambda i, j: (j, 0)),
        ],
        out_specs=pl.BlockSpec((bi, bi), lambda i, j: (i, j)),
        out_shape=jax.ShapeDtypeStruct((N, N), jnp.float32),
    )(embed, embed)


def _tacc_body(x_ref, wa1_ref, acc_ref):
    @pl.when(pl.program_id(0) == 0)
    def _():
        acc_ref[...] = jnp.zeros_like(acc_ref)

    acc_ref[...] += lax.dot_general(
        x_ref[...], wa1_ref[...], (((0,), (0,)), ((), ())),
        preferred_element_type=jnp.float32,
    )


def _tacc(x, Wa1):
    grid = (N // BN,)
    return pl.pallas_call(
        _tacc_body,
        grid=grid,
        in_specs=[
            pl.BlockSpec((BN, D), lambda i: (i, 0)),
            pl.BlockSpec((BN, EMB), lambda i: (i, 0)),
        ],
        out_specs=pl.BlockSpec((D, EMB), lambda i: (0, 0)),
        out_shape=jax.ShapeDtypeStruct((D, EMB), jnp.float32),
    )(x, Wa1)


def kernel(x, edge_index, W1, b1, Wg, a_src, a_dst, bg, Wa1, ba1, Wa2, ba2):
    a2 = jnp.stack([a_src, a_dst], axis=1)
    hw_aug, s2 = _encoder(x, W1, b1, Wg, a2)
    tacc = _tacc(x, Wa1)
    pad = NCHP * CH - E
    src2d = jnp.pad(edge_index[0], (0, pad)).reshape(NCHP, CH)
    dst2d = jnp.pad(edge_index[1], (0, pad)).reshape(NCHP, CH)
    acc2 = _sc_edge(hw_aug, s2[:, 0], s2[:, 1], src2d, dst2d)
    embed, X_hat = _finalize_xhat(acc2, bg, tacc, ba1, Wa2, ba2)
    A_hat = _ahat(embed)
    return (A_hat, X_hat)


# speedup vs baseline: 1.0202x; 1.0099x over previous
import functools

import jax
import jax.numpy as jnp
from jax import lax
from jax.experimental import pallas as pl
from jax.experimental.pallas import tpu as pltpu
from jax.experimental.pallas import tpu_sc as plsc

N = 10000
E = 160000
D = 256
EMB = 256
OUT = 128

BN = 2000

HWA = 144
CH = 64
NCH = E // CH
CPT = 80
NCHP = 32 * CPT
NSUB = 16
ZR = 624


def _enc_body(x_ref, w1_ref, b1_ref, a2_ref, wg_ref, hw_ref, s2_ref):
    x = x_ref[...]
    h = jnp.maximum(
        jnp.dot(x, w1_ref[...], preferred_element_type=jnp.float32)
        + b1_ref[...],
        0.0,
    )
    hw = jnp.dot(h, wg_ref[...], preferred_element_type=jnp.float32)
    pad_cols = lax.broadcasted_iota(jnp.int32, (BN, HWA - OUT), 1)
    pad = jnp.where(pad_cols == 0, 1.0, 0.0).astype(jnp.float32)
    hw_ref[...] = jnp.concatenate([hw, pad], axis=1)
    s2_ref[...] = jnp.dot(hw, a2_ref[...], preferred_element_type=jnp.float32)


def _encoder(x, W1, b1, Wg, a2):
    grid = (N // BN,)
    return pl.pallas_call(
        _enc_body,
        grid=grid,
        in_specs=[
            pl.BlockSpec((BN, D), lambda i: (i, 0)),
            pl.BlockSpec((D, EMB), lambda i: (0, 0)),
            pl.BlockSpec((1, EMB), lambda i: (0, 0)),
            pl.BlockSpec((OUT, 2), lambda i: (0, 0)),
            pl.BlockSpec((EMB, OUT), lambda i: (0, 0)),
        ],
        out_specs=[
            pl.BlockSpec((BN, HWA), lambda i: (i, 0)),
            pl.BlockSpec((BN, 2), lambda i: (i, 0)),
        ],
        out_shape=[
            jax.ShapeDtypeStruct((N, HWA), jnp.float32),
            jax.ShapeDtypeStruct((N, 2), jnp.float32),
        ],
    )(x, W1, b1.reshape(1, EMB), a2, Wg)


def _sc_edge(hw_aug, s_src, s_dst, src2d, dst2d):
    mesh = plsc.VectorSubcoreMesh(core_axis_name="c", subcore_axis_name="s")

    @functools.partial(
        pl.kernel,
        mesh=mesh,
        out_type=jax.ShapeDtypeStruct((2, N, HWA), jnp.float32),
        scratch_types=[
            pltpu.VMEM((CPT, CH), jnp.int32),
            pltpu.VMEM((CPT, CH), jnp.int32),
            pltpu.VMEM((3, CH), jnp.float32),
            pltpu.VMEM((3, CH), jnp.float32),
            pltpu.VMEM((3, CH + 16), jnp.float32),
            pltpu.VMEM((3, CH, HWA), jnp.float32),
            pltpu.VMEM_SHARED((N, HWA), jnp.float32),
            pltpu.SemaphoreType.DMA,
            pltpu.SemaphoreType.DMA,
            pltpu.SemaphoreType.DMA,
            pltpu.SemaphoreType.DMA,
            pltpu.SemaphoreType.DMA,
            pltpu.SemaphoreType.DMA,
            pltpu.SemaphoreType.DMA,
            pltpu.SemaphoreType.DMA,
            pltpu.SemaphoreType.DMA,
        ],
        compiler_params=pltpu.CompilerParams(use_tc_tiling_on_sc=False),
    )
    def k(hw_hbm, ssrc_hbm, sdst_hbm, src_hbm, dst_hbm, out_hbm,
          src_idx, dst_idx, ssb, sdb, exb, rows, acc_sh,
          sem_r0, sem_r1, sem_r2, sem_s0, sem_s1, sem_s2,
          sem_w0, sem_w1, sem_w2):
        c = lax.axis_index("c")
        s = lax.axis_index("s")
        w = c * NSUB + s

        def zrow(r, _):
            for kk in range(HWA // 16):
                rows[0, r, pl.ds(kk * 16, 16)] = jnp.zeros((16,), jnp.float32)
            return 0
        lax.fori_loop(0, CH, zrow, 0)

        pltpu.async_copy(src_hbm.at[pl.ds(w * CPT, CPT)], src_idx, sem_w1)
        pltpu.async_copy(dst_hbm.at[pl.ds(w * CPT, CPT)], dst_idx, sem_w1)

        def zcp(i, _):
            pltpu.async_copy(rows.at[0].at[pl.ds(0, 48)],
                             acc_sh.at[pl.ds(s * ZR + i * 48, 48)], sem_w0)
            return 0
        lax.fori_loop(0, ZR // 48, zcp, 0)

        @pl.when(s == 0)
        def _():
            pltpu.sync_copy(rows.at[0].at[pl.ds(0, 16)],
                            acc_sh.at[pl.ds(N - 16, 16)])

        def zcw(i, _):
            pltpu.make_async_copy(rows.at[0].at[pl.ds(0, 48)],
                                  acc_sh.at[pl.ds(s * ZR + i * 48, 48)],
                                  sem_w0).wait()
            return 0
        lax.fori_loop(0, ZR // 48, zcw, 0)

        pltpu.make_async_copy(src_hbm.at[pl.ds(w * CPT, CPT)], src_idx,
                              sem_w1).wait()
        pltpu.make_async_copy(dst_hbm.at[pl.ds(w * CPT, CPT)], dst_idx,
                              sem_w1).wait()
        n_real = jnp.clip(NCH - w * CPT, 0, CPT)

        sem_r = [sem_r0, sem_r1, sem_r2]
        sem_s = [sem_s0, sem_s1, sem_s2]
        sem_w = [sem_w0, sem_w1, sem_w2]

        def issue(j, p):
            pltpu.async_copy(ssrc_hbm.at[src_idx.at[j]], ssb.at[p], sem_s[p])
            pltpu.async_copy(sdst_hbm.at[dst_idx.at[j]], sdb.at[p], sem_s[p])
            pltpu.async_copy(hw_hbm.at[src_idx.at[j]], rows.at[p], sem_r[p])

        def wait(j, p):
            pltpu.make_async_copy(ssrc_hbm.at[src_idx.at[j]], ssb.at[p],
                                  sem_s[p]).wait()
            pltpu.make_async_copy(sdst_hbm.at[dst_idx.at[j]], sdb.at[p],
                                  sem_s[p]).wait()
            pltpu.make_async_copy(hw_hbm.at[src_idx.at[j]], rows.at[p],
                                  sem_r[p]).wait()

        plsc.subcore_barrier()

        @pl.when(n_real > 0)
        def _():
            issue(0, 0)

        def process(j, p):
            @pl.when(j >= 2)
            def _():
                pltpu.make_async_copy(
                    rows.at[(p + 1) % 3], acc_sh.at[dst_idx.at[j - 2]],
                    sem_w[(p + 1) % 3]).wait()

            @pl.when(j + 1 < n_real)
            def _():
                issue(j + 1, (p + 1) % 3)

            wait(j, p)

            for g in range(CH // 16):
                sl = pl.ds(g * 16, 16)
                logit = ssb[p, sl] + sdb[p, sl]
                logit = jnp.where(logit >= 0, logit, 0.2 * logit)
                exb[p, sl] = jnp.exp(logit)

            @plsc.parallel_loop(0, CH, 1, unroll=8)
            def _(r):
                e = exb[p, pl.ds(r, 16)][0]
                for kk in range(HWA // 16):
                    sl = pl.ds(kk * 16, 16)
                    rows[p, r, sl] = rows[p, r, sl] * e

            @pl.when(j + 2 < n_real)
            def _():
                pltpu.async_copy(rows.at[p], acc_sh.at[dst_idx.at[j]],
                                 sem_w[p], add=True)

            @pl.when(j + 2 >= n_real)
            def _():
                pltpu.sync_copy(rows.at[p], acc_sh.at[dst_idx.at[j]], add=True)

        def chunk3(j3, _):
            for phase in range(3):
                j = j3 * 3 + phase

                @pl.when(j < n_real)
                def _():
                    process(j, phase)
            return 0
        lax.fori_loop(0, (n_real + 2) // 3, chunk3, 0)

        plsc.subcore_barrier()

        def wout(i, _):
            sl = pl.ds(s * ZR + i * 48, 48)
            pltpu.async_copy(acc_sh.at[sl], out_hbm.at[c].at[sl], sem_w0)
            return 0
        lax.fori_loop(0, ZR // 48, wout, 0)

        @pl.when(s == 0)
        def _():
            sl = pl.ds(N - 16, 16)
            pltpu.sync_copy(acc_sh.at[sl], out_hbm.at[c].at[sl])

        def woutw(i, _):
            sl = pl.ds(s * ZR + i * 48, 48)
            pltpu.make_async_copy(acc_sh.at[sl], out_hbm.at[c].at[sl],
                                  sem_w0).wait()
            return 0
        lax.fori_loop(0, ZR // 48, woutw, 0)

    return k(hw_aug, s_src, s_dst, src2d, dst2d)


def _finx_body(acc_ref, bg_ref, tacc_ref, ba1_ref, wa2_ref, ba2_ref,
               emb_ref, xhat_ref):
    a = acc_ref[0] + acc_ref[1]
    num = a[:, :OUT]
    den = a[:, OUT:OUT + 1]
    embed = num / (den + 1e-16) + bg_ref[...]
    emb_ref[...] = embed
    t1 = jnp.maximum(tacc_ref[...] + ba1_ref[...], 0.0)
    t2 = jnp.dot(t1, wa2_ref[...], preferred_element_type=jnp.float32)
    t2 = t2 + ba2_ref[...]
    xhat_ref[...] = lax.dot_general(
        embed, t2, (((1,), (1,)), ((), ())),
        preferred_element_type=jnp.float32,
    )


def _finalize_xhat(acc2, bg, tacc, ba1, Wa2, ba2):
    grid = (N // BN,)
    return pl.pallas_call(
        _finx_body,
        grid=grid,
        in_specs=[
            pl.BlockSpec((2, BN, HWA), lambda i: (0, i, 0)),
            pl.BlockSpec((1, OUT), lambda i: (0, 0)),
            pl.BlockSpec((D, EMB), lambda i: (0, 0)),
            pl.BlockSpec((1, EMB), lambda i: (0, 0)),
            pl.BlockSpec((EMB, OUT), lambda i: (0, 0)),
            pl.BlockSpec((1, OUT), lambda i: (0, 0)),
        ],
        out_specs=[
            pl.BlockSpec((BN, OUT), lambda i: (i, 0)),
            pl.BlockSpec((BN, D), lambda i: (i, 0)),
        ],
        out_shape=[
            jax.ShapeDtypeStruct((N, OUT), jnp.float32),
            jax.ShapeDtypeStruct((N, D), jnp.float32),
        ],
    )(acc2, bg.reshape(1, OUT), tacc, ba1.reshape(1, EMB), Wa2,
      ba2.reshape(1, OUT))


def _ahat_body(ei_ref, ej_ref, out_ref):
    acc = lax.dot_general(
        ei_ref[...], ej_ref[...], (((1,), (1,)), ((), ())),
        preferred_element_type=jnp.float32,
    )
    out_ref[...] = jax.nn.sigmoid(acc)


def _ahat(embed):
    bi = 2048
    nb = (N + bi - 1) // bi
    grid = (nb, nb)
    return pl.pallas_call(
        _ahat_body,
        grid=grid,
        in_specs=[
            pl.BlockSpec((bi, OUT), lambda i, j: (i, 0)),
            pl.BlockSpec((bi, OUT), lambda i, j: (j, 0)),
        ],
        out_specs=pl.BlockSpec((bi, bi), lambda i, j: (i, j)),
        out_shape=jax.ShapeDtypeStruct((N, N), jnp.float32),
    )(embed, embed)


def _tacc_body(x_ref, wa1_ref, acc_ref):
    @pl.when(pl.program_id(0) == 0)
    def _():
        acc_ref[...] = jnp.zeros_like(acc_ref)

    acc_ref[...] += lax.dot_general(
        x_ref[...], wa1_ref[...], (((0,), (0,)), ((), ())),
        preferred_element_type=jnp.float32,
    )


def _tacc(x, Wa1):
    grid = (N // BN,)
    return pl.pallas_call(
        _tacc_body,
        grid=grid,
        in_specs=[
            pl.BlockSpec((BN, D), lambda i: (i, 0)),
            pl.BlockSpec((BN, EMB), lambda i: (i, 0)),
        ],
        out_specs=pl.BlockSpec((D, EMB), lambda i: (0, 0)),
        out_shape=jax.ShapeDtypeStruct((D, EMB), jnp.float32),
    )(x, Wa1)


def kernel(x, edge_index, W1, b1, Wg, a_src, a_dst, bg, Wa1, ba1, Wa2, ba2):
    a2 = jnp.stack([a_src, a_dst], axis=1)
    hw_aug, s2 = _encoder(x, W1, b1, Wg, a2)
    tacc = _tacc(x, Wa1)
    pad = NCHP * CH - E
    src2d = jnp.pad(edge_index[0], (0, pad)).reshape(NCHP, CH)
    dst2d = jnp.pad(edge_index[1], (0, pad)).reshape(NCHP, CH)
    acc2 = _sc_edge(hw_aug, s2[:, 0], s2[:, 1], src2d, dst2d)
    embed, X_hat = _finalize_xhat(acc2, bg, tacc, ba1, Wa2, ba2)
    A_hat = _ahat(embed)
    return (A_hat, X_hat)


# speedup vs baseline: 1.0225x; 1.0023x over previous
import functools

import jax
import jax.numpy as jnp
from jax import lax
from jax.experimental import pallas as pl
from jax.experimental.pallas import tpu as pltpu
from jax.experimental.pallas import tpu_sc as plsc

N = 10000
E = 160000
D = 256
EMB = 256
OUT = 128

BN = 2000

HWA = 144
CH = 64
NCH = E // CH
CPT = 80
NCHP = 32 * CPT
NSUB = 16
ZR = 624


def _enc_body(x_ref, w1_ref, b1_ref, a2_ref, wg_ref, hw_ref, s2_ref):
    x = x_ref[...]
    h = jnp.maximum(
        jnp.dot(x, w1_ref[...], preferred_element_type=jnp.float32)
        + b1_ref[...],
        0.0,
    )
    hw = jnp.dot(h, wg_ref[...], preferred_element_type=jnp.float32)
    pad_cols = lax.broadcasted_iota(jnp.int32, (BN, HWA - OUT), 1)
    pad = jnp.where(pad_cols == 0, 1.0, 0.0).astype(jnp.float32)
    hw_ref[...] = jnp.concatenate([hw, pad], axis=1)
    s2_ref[...] = jnp.dot(hw, a2_ref[...], preferred_element_type=jnp.float32)


def _encoder(x, W1, b1, Wg, a2):
    grid = (N // BN,)
    return pl.pallas_call(
        _enc_body,
        grid=grid,
        in_specs=[
            pl.BlockSpec((BN, D), lambda i: (i, 0)),
            pl.BlockSpec((D, EMB), lambda i: (0, 0)),
            pl.BlockSpec((1, EMB), lambda i: (0, 0)),
            pl.BlockSpec((OUT, 2), lambda i: (0, 0)),
            pl.BlockSpec((EMB, OUT), lambda i: (0, 0)),
        ],
        out_specs=[
            pl.BlockSpec((BN, HWA), lambda i: (i, 0)),
            pl.BlockSpec((BN, 2), lambda i: (i, 0)),
        ],
        out_shape=[
            jax.ShapeDtypeStruct((N, HWA), jnp.float32),
            jax.ShapeDtypeStruct((N, 2), jnp.float32),
        ],
    )(x, W1, b1.reshape(1, EMB), a2, Wg)


def _sc_edge(hw_aug, s_src, s_dst, src2d, dst2d):
    mesh = plsc.VectorSubcoreMesh(core_axis_name="c", subcore_axis_name="s")

    @functools.partial(
        pl.kernel,
        mesh=mesh,
        out_type=jax.ShapeDtypeStruct((2, N, HWA), jnp.float32),
        scratch_types=[
            pltpu.VMEM((CPT, CH), jnp.int32),
            pltpu.VMEM((CPT, CH), jnp.int32),
            pltpu.VMEM((3, CH), jnp.float32),
            pltpu.VMEM((3, CH), jnp.float32),
            pltpu.VMEM((3, CH + 16), jnp.float32),
            pltpu.VMEM((3, CH, HWA), jnp.float32),
            pltpu.VMEM_SHARED((N, HWA), jnp.float32),
            pltpu.SemaphoreType.DMA,
            pltpu.SemaphoreType.DMA,
            pltpu.SemaphoreType.DMA,
            pltpu.SemaphoreType.DMA,
            pltpu.SemaphoreType.DMA,
            pltpu.SemaphoreType.DMA,
            pltpu.SemaphoreType.DMA,
            pltpu.SemaphoreType.DMA,
            pltpu.SemaphoreType.DMA,
        ],
        compiler_params=pltpu.CompilerParams(use_tc_tiling_on_sc=False),
    )
    def k(hw_hbm, ssrc_hbm, sdst_hbm, src_hbm, dst_hbm, out_hbm,
          src_idx, dst_idx, ssb, sdb, exb, rows, acc_sh,
          sem_r0, sem_r1, sem_r2, sem_s0, sem_s1, sem_s2,
          sem_w0, sem_w1, sem_w2):
        c = lax.axis_index("c")
        s = lax.axis_index("s")
        w = c * NSUB + s

        def zrow(r, _):
            for kk in range(HWA // 16):
                rows[0, r, pl.ds(kk * 16, 16)] = jnp.zeros((16,), jnp.float32)
            return 0
        lax.fori_loop(0, CH, zrow, 0)

        pltpu.async_copy(src_hbm.at[pl.ds(w * CPT, CPT)], src_idx, sem_w1)
        pltpu.async_copy(dst_hbm.at[pl.ds(w * CPT, CPT)], dst_idx, sem_w1)

        def zcp(i, _):
            pltpu.async_copy(rows.at[0].at[pl.ds(0, 48)],
                             acc_sh.at[pl.ds(s * ZR + i * 48, 48)], sem_w0)
            return 0
        lax.fori_loop(0, ZR // 48, zcp, 0)

        @pl.when(s == 0)
        def _():
            pltpu.sync_copy(rows.at[0].at[pl.ds(0, 16)],
                            acc_sh.at[pl.ds(N - 16, 16)])

        def zcw(i, _):
            pltpu.make_async_copy(rows.at[0].at[pl.ds(0, 48)],
                                  acc_sh.at[pl.ds(s * ZR + i * 48, 48)],
                                  sem_w0).wait()
            return 0
        lax.fori_loop(0, ZR // 48, zcw, 0)

        pltpu.make_async_copy(src_hbm.at[pl.ds(w * CPT, CPT)], src_idx,
                              sem_w1).wait()
        pltpu.make_async_copy(dst_hbm.at[pl.ds(w * CPT, CPT)], dst_idx,
                              sem_w1).wait()
        n_real = jnp.clip(NCH - w * CPT, 0, CPT)

        sem_r = [sem_r0, sem_r1, sem_r2]
        sem_s = [sem_s0, sem_s1, sem_s2]
        sem_w = [sem_w0, sem_w1, sem_w2]

        def issue(j, p):
            pltpu.async_copy(ssrc_hbm.at[src_idx.at[j]], ssb.at[p], sem_s[p])
            pltpu.async_copy(sdst_hbm.at[dst_idx.at[j]], sdb.at[p], sem_s[p])
            pltpu.async_copy(hw_hbm.at[src_idx.at[j]], rows.at[p], sem_r[p])

        def wait_s(j, p):
            pltpu.make_async_copy(ssrc_hbm.at[src_idx.at[j]], ssb.at[p],
                                  sem_s[p]).wait()
            pltpu.make_async_copy(sdst_hbm.at[dst_idx.at[j]], sdb.at[p],
                                  sem_s[p]).wait()

        def wait_rows(j, p):
            pltpu.make_async_copy(hw_hbm.at[src_idx.at[j]], rows.at[p],
                                  sem_r[p]).wait()

        plsc.subcore_barrier()

        @pl.when(n_real > 0)
        def _():
            issue(0, 0)

        def process(j, p):
            @pl.when(j >= 2)
            def _():
                pltpu.make_async_copy(
                    rows.at[(p + 1) % 3], acc_sh.at[dst_idx.at[j - 2]],
                    sem_w[(p + 1) % 3]).wait()

            @pl.when(j + 1 < n_real)
            def _():
                issue(j + 1, (p + 1) % 3)

            wait_s(j, p)

            for g in range(CH // 16):
                sl = pl.ds(g * 16, 16)
                logit = ssb[p, sl] + sdb[p, sl]
                logit = jnp.where(logit >= 0, logit, 0.2 * logit)
                exb[p, sl] = jnp.exp(logit)

            wait_rows(j, p)

            @plsc.parallel_loop(0, CH, 1, unroll=8)
            def _(r):
                e = exb[p, pl.ds(r, 16)][0]
                for kk in range(HWA // 16):
                    sl = pl.ds(kk * 16, 16)
                    rows[p, r, sl] = rows[p, r, sl] * e

            @pl.when(j + 2 < n_real)
            def _():
                pltpu.async_copy(rows.at[p], acc_sh.at[dst_idx.at[j]],
                                 sem_w[p], add=True)

            @pl.when(j + 2 >= n_real)
            def _():
                pltpu.sync_copy(rows.at[p], acc_sh.at[dst_idx.at[j]], add=True)

        def chunk3(j3, _):
            for phase in range(3):
                j = j3 * 3 + phase

                @pl.when(j < n_real)
                def _():
                    process(j, phase)
            return 0
        lax.fori_loop(0, (n_real + 2) // 3, chunk3, 0)

        plsc.subcore_barrier()

        def wout(i, _):
            sl = pl.ds(s * ZR + i * 48, 48)
            pltpu.async_copy(acc_sh.at[sl], out_hbm.at[c].at[sl], sem_w0)
            return 0
        lax.fori_loop(0, ZR // 48, wout, 0)

        @pl.when(s == 0)
        def _():
            sl = pl.ds(N - 16, 16)
            pltpu.sync_copy(acc_sh.at[sl], out_hbm.at[c].at[sl])

        def woutw(i, _):
            sl = pl.ds(s * ZR + i * 48, 48)
            pltpu.make_async_copy(acc_sh.at[sl], out_hbm.at[c].at[sl],
                                  sem_w0).wait()
            return 0
        lax.fori_loop(0, ZR // 48, woutw, 0)

    return k(hw_aug, s_src, s_dst, src2d, dst2d)


def _finx_body(acc_ref, bg_ref, tacc_ref, ba1_ref, wa2_ref, ba2_ref,
               emb_ref, xhat_ref):
    a = acc_ref[0] + acc_ref[1]
    num = a[:, :OUT]
    den = a[:, OUT:OUT + 1]
    embed = num / (den + 1e-16) + bg_ref[...]
    emb_ref[...] = embed
    t1 = jnp.maximum(tacc_ref[...] + ba1_ref[...], 0.0)
    t2 = jnp.dot(t1, wa2_ref[...], preferred_element_type=jnp.float32)
    t2 = t2 + ba2_ref[...]
    xhat_ref[...] = lax.dot_general(
        embed, t2, (((1,), (1,)), ((), ())),
        preferred_element_type=jnp.float32,
    )


def _finalize_xhat(acc2, bg, tacc, ba1, Wa2, ba2):
    grid = (N // BN,)
    return pl.pallas_call(
        _finx_body,
        grid=grid,
        in_specs=[
            pl.BlockSpec((2, BN, HWA), lambda i: (0, i, 0)),
            pl.BlockSpec((1, OUT), lambda i: (0, 0)),
            pl.BlockSpec((D, EMB), lambda i: (0, 0)),
            pl.BlockSpec((1, EMB), lambda i: (0, 0)),
            pl.BlockSpec((EMB, OUT), lambda i: (0, 0)),
            pl.BlockSpec((1, OUT), lambda i: (0, 0)),
        ],
        out_specs=[
            pl.BlockSpec((BN, OUT), lambda i: (i, 0)),
            pl.BlockSpec((BN, D), lambda i: (i, 0)),
        ],
        out_shape=[
            jax.ShapeDtypeStruct((N, OUT), jnp.float32),
            jax.ShapeDtypeStruct((N, D), jnp.float32),
        ],
    )(acc2, bg.reshape(1, OUT), tacc, ba1.reshape(1, EMB), Wa2,
      ba2.reshape(1, OUT))


def _ahat_body(ei_ref, ej_ref, out_ref):
    acc = lax.dot_general(
        ei_ref[...], ej_ref[...], (((1,), (1,)), ((), ())),
        preferred_element_type=jnp.float32,
    )
    out_ref[...] = jax.nn.sigmoid(acc)


def _ahat(embed):
    bi = 2048
    nb = (N + bi - 1) // bi
    grid = (nb, nb)
    return pl.pallas_call(
        _ahat_body,
        grid=grid,
        in_specs=[
            pl.BlockSpec((bi, OUT), lambda i, j: (i, 0)),
            pl.BlockSpec((bi, OUT), lambda i, j: (j, 0)),
        ],
        out_specs=pl.BlockSpec((bi, bi), lambda i, j: (i, j)),
        out_shape=jax.ShapeDtypeStruct((N, N), jnp.float32),
    )(embed, embed)


def _tacc_body(x_ref, wa1_ref, acc_ref):
    @pl.when(pl.program_id(0) == 0)
    def _():
        acc_ref[...] = jnp.zeros_like(acc_ref)

    acc_ref[...] += lax.dot_general(
        x_ref[...], wa1_ref[...], (((0,), (0,)), ((), ())),
        preferred_element_type=jnp.float32,
    )


def _tacc(x, Wa1):
    grid = (N // BN,)
    return pl.pallas_call(
        _tacc_body,
        grid=grid,
        in_specs=[
            pl.BlockSpec((BN, D), lambda i: (i, 0)),
            pl.BlockSpec((BN, EMB), lambda i: (i, 0)),
        ],
        out_specs=pl.BlockSpec((D, EMB), lambda i: (0, 0)),
        out_shape=jax.ShapeDtypeStruct((D, EMB), jnp.float32),
    )(x, Wa1)


def kernel(x, edge_index, W1, b1, Wg, a_src, a_dst, bg, Wa1, ba1, Wa2, ba2):
    a2 = jnp.stack([a_src, a_dst], axis=1)
    hw_aug, s2 = _encoder(x, W1, b1, Wg, a2)
    tacc = _tacc(x, Wa1)
    pad = NCHP * CH - E
    src2d = jnp.pad(edge_index[0], (0, pad)).reshape(NCHP, CH)
    dst2d = jnp.pad(edge_index[1], (0, pad)).reshape(NCHP, CH)
    acc2 = _sc_edge(hw_aug, s2[:, 0], s2[:, 1], src2d, dst2d)
    embed, X_hat = _finalize_xhat(acc2, bg, tacc, ba1, Wa2, ba2)
    A_hat = _ahat(embed)
    return (A_hat, X_hat)
